# baseline (speedup 1.0000x reference)
import dataclasses

import jax
import jax.numpy as jnp
from jax import lax
from jax.experimental import pallas as pl
from jax.experimental.pallas import tpu as pltpu
from jax.experimental.pallas import tpu_sc as plsc

_LANES = 16
_NUM_WORKERS = 32
_PADK = 1024
_UNROLL = 1
_NBUCKET = 256
_SHIFT = 19
_BUCKET_OFF = 1923
_ASCENT = (8, 4, 2, 1)


def _sigma_to_t_kernel(q_hbm, tab_hbm, out_hbm, ftab_v, lut_v,
                       q_v, o_v, sem_f, sem_q):
    nq = q_hbm.shape[0]
    k_levels = tab_hbm.shape[0]
    per_worker = nq // _NUM_WORKERS
    wid = lax.axis_index("s") * 2 + lax.axis_index("c")
    base = wid * per_worker

    cp_f = pltpu.async_copy(tab_hbm, ftab_v.at[pl.ds(0, k_levels)], sem_f)
    cp_q = pltpu.async_copy(q_hbm.at[pl.ds(base, per_worker)], q_v, sem_q)

    def bucket_of(vals_f32):
        b = lax.shift_right_arithmetic(
            lax.bitcast_convert_type(vals_f32, jnp.int32), _SHIFT)
        b = b - _BUCKET_OFF
        return jnp.clip(b, 0, _NBUCKET - 1)

    zeros16 = jnp.zeros((_LANES,), jnp.int32)

    @plsc.parallel_loop(0, _NBUCKET, step=_LANES)
    def _(i):
        lut_v[pl.ds(i, _LANES)] = zeros16

    cp_f.wait()
    sent = jnp.full((_LANES,), -1.0, jnp.float32)
    ftab_v[pl.ds(k_levels, _LANES)] = sent
    ftab_v[pl.ds(_PADK - _LANES, _LANES)] = sent

    ones16 = jnp.ones((_LANES,), jnp.int32)
    full_chunks = ((k_levels - 1) // _LANES) * _LANES

    @plsc.parallel_loop(0, full_chunks, step=_LANES)
    def _(i):
        bb = bucket_of(ftab_v[pl.ds(i, _LANES)])
        plsc.addupdate_scatter(lut_v, [bb], ones16)

    tail = (k_levels - 1) - full_chunks
    if tail:
        bb = bucket_of(ftab_v[pl.ds(full_chunks, _LANES)])
        tmask = lax.iota(jnp.int32, _LANES) < tail
        plsc.addupdate_scatter(lut_v, [bb], ones16, mask=tmask)

    def _suffix_body(t, carry):
        i = _NBUCKET - _LANES - t * _LANES
        h = lut_v[pl.ds(i, _LANES)]
        cs = plsc.cumsum(lax.rev(h, (0,))) + carry
        new_carry = lax.squeeze(lax.slice(cs, (_LANES - 1,), (_LANES,)), (0,))
        lut_v[pl.ds(i, _LANES)] = jnp.clip(lax.rev(cs, (0,)) - 1,
                                           0, k_levels - 2)
        return new_carry

    lax.fori_loop(0, _NBUCKET // _LANES, _suffix_body, jnp.int32(0))

    cp_q.wait()

    def _query_block(lo):
        @plsc.parallel_loop(lo, lo + per_worker, step=_LANES * _UNROLL)
        def _(i):
            qs = [q_v[pl.ds(i + u * _LANES, _LANES)] for u in range(_UNROLL)]
            js = [plsc.load_gather(
                      lut_v,
                      [jnp.clip(lax.shift_right_arithmetic(
                           lax.bitcast_convert_type(qs[u], jnp.int32), _SHIFT)
                           - (_BUCKET_OFF - 1), 0, _NBUCKET - 1)])
                  for u in range(_UNROLL)]
            for step in _ASCENT:
                for u in range(_UNROLL):
                    cand = js[u] + step
                    val = plsc.load_gather(ftab_v, [cand])
                    js[u] = jnp.where(val >= qs[u], cand, js[u])
            for u in range(_UNROLL):
                j, q = jnp.minimum(js[u], k_levels - 2), qs[u]
                low = plsc.load_gather(ftab_v, [j])
                high = plsc.load_gather(ftab_v, [j + 1])
                w = jnp.clip((low - q) / (low - high), 0.0, 1.0)
                jf = j.astype(jnp.float32)
                o_v[pl.ds(i + u * _LANES, _LANES)] = ((1.0 - w) * jf
                                                     + w * (jf + 1.0))

    _query_block(0)
    pltpu.sync_copy(o_v, out_hbm.at[pl.ds(base, per_worker)])


@jax.jit
def kernel(sigma, sigmas):
    nq = sigma.shape[0]
    per_worker = nq // _NUM_WORKERS

    mesh = plsc.VectorSubcoreMesh(core_axis_name="c", subcore_axis_name="s")
    cp = pltpu.CompilerParams()
    if "needs_layout_passes" in pltpu.CompilerParams.__dataclass_fields__:
        cp = dataclasses.replace(cp, needs_layout_passes=False)
    run = pl.kernel(
        _sigma_to_t_kernel,
        out_type=jax.ShapeDtypeStruct((nq,), jnp.float32),
        mesh=mesh,
        scratch_types=[
            pltpu.VMEM((_PADK,), jnp.float32),
            pltpu.VMEM((_NBUCKET,), jnp.int32),
            pltpu.VMEM((per_worker,), jnp.float32),
            pltpu.VMEM((per_worker,), jnp.float32),
            pltpu.SemaphoreType.DMA,
            pltpu.SemaphoreType.DMA,
        ],
        compiler_params=cp,
    )
    return run(sigma, sigmas).reshape(sigma.shape)


# speedup vs baseline: 1.0041x; 1.0041x over previous
import dataclasses

import jax
import jax.numpy as jnp
from jax import lax
from jax.experimental import pallas as pl
from jax.experimental.pallas import tpu as pltpu
from jax.experimental.pallas import tpu_sc as plsc

_LANES = 16
_NUM_WORKERS = 32
_PADK = 1024
_UNROLL = 1
_NBUCKET = 128
_SHIFT = 20
_BUCKET_OFF = 961
_ASCENT = (16, 8, 4, 2, 1)


def _sigma_to_t_kernel(q_hbm, tab_hbm, out_hbm, ftab_v, lut_v,
                       q_v, o_v, sem_f, sem_q):
    nq = q_hbm.shape[0]
    k_levels = tab_hbm.shape[0]
    per_worker = nq // _NUM_WORKERS
    wid = lax.axis_index("s") * 2 + lax.axis_index("c")
    base = wid * per_worker

    cp_f = pltpu.async_copy(tab_hbm, ftab_v.at[pl.ds(0, k_levels)], sem_f)
    cp_q = pltpu.async_copy(q_hbm.at[pl.ds(base, per_worker)], q_v, sem_q)

    def bucket_of(vals_f32):
        b = lax.shift_right_arithmetic(
            lax.bitcast_convert_type(vals_f32, jnp.int32), _SHIFT)
        b = b - _BUCKET_OFF
        return jnp.clip(b, 0, _NBUCKET - 1)

    zeros16 = jnp.zeros((_LANES,), jnp.int32)

    @plsc.parallel_loop(0, _NBUCKET, step=_LANES)
    def _(i):
        lut_v[pl.ds(i, _LANES)] = zeros16

    cp_f.wait()
    sent = jnp.full((_LANES,), -1.0, jnp.float32)
    ftab_v[pl.ds(k_levels, _LANES)] = sent
    ftab_v[pl.ds(_PADK - _LANES, _LANES)] = sent

    ones16 = jnp.ones((_LANES,), jnp.int32)
    full_chunks = ((k_levels - 1) // _LANES) * _LANES

    @plsc.parallel_loop(0, full_chunks, step=_LANES)
    def _(i):
        bb = bucket_of(ftab_v[pl.ds(i, _LANES)])
        plsc.addupdate_scatter(lut_v, [bb], ones16)

    tail = (k_levels - 1) - full_chunks
    if tail:
        bb = bucket_of(ftab_v[pl.ds(full_chunks, _LANES)])
        tmask = lax.iota(jnp.int32, _LANES) < tail
        plsc.addupdate_scatter(lut_v, [bb], ones16, mask=tmask)

    def _suffix_body(t, carry):
        i = _NBUCKET - _LANES - t * _LANES
        h = lut_v[pl.ds(i, _LANES)]
        cs = plsc.cumsum(lax.rev(h, (0,))) + carry
        new_carry = lax.squeeze(lax.slice(cs, (_LANES - 1,), (_LANES,)), (0,))
        lut_v[pl.ds(i, _LANES)] = jnp.clip(lax.rev(cs, (0,)) - 1,
                                           0, k_levels - 2)
        return new_carry

    lax.fori_loop(0, _NBUCKET // _LANES, _suffix_body, jnp.int32(0))

    cp_q.wait()

    def _query_block(lo):
        @plsc.parallel_loop(lo, lo + per_worker, step=_LANES * _UNROLL)
        def _(i):
            qs = [q_v[pl.ds(i + u * _LANES, _LANES)] for u in range(_UNROLL)]
            js = [plsc.load_gather(
                      lut_v,
                      [jnp.clip(lax.shift_right_arithmetic(
                           lax.bitcast_convert_type(qs[u], jnp.int32), _SHIFT)
                           - (_BUCKET_OFF - 1), 0, _NBUCKET - 1)])
                  for u in range(_UNROLL)]
            for step in _ASCENT:
                for u in range(_UNROLL):
                    cand = js[u] + step
                    val = plsc.load_gather(ftab_v, [cand])
                    js[u] = jnp.where(val >= qs[u], cand, js[u])
            for u in range(_UNROLL):
                j, q = jnp.minimum(js[u], k_levels - 2), qs[u]
                low = plsc.load_gather(ftab_v, [j])
                high = plsc.load_gather(ftab_v, [j + 1])
                w = jnp.clip((low - q) / (low - high), 0.0, 1.0)
                jf = j.astype(jnp.float32)
                o_v[pl.ds(i + u * _LANES, _LANES)] = ((1.0 - w) * jf
                                                     + w * (jf + 1.0))

    _query_block(0)
    pltpu.sync_copy(o_v, out_hbm.at[pl.ds(base, per_worker)])


@jax.jit
def kernel(sigma, sigmas):
    nq = sigma.shape[0]
    per_worker = nq // _NUM_WORKERS

    mesh = plsc.VectorSubcoreMesh(core_axis_name="c", subcore_axis_name="s")
    cp = pltpu.CompilerParams()
    if "needs_layout_passes" in pltpu.CompilerParams.__dataclass_fields__:
        cp = dataclasses.replace(cp, needs_layout_passes=False)
    run = pl.kernel(
        _sigma_to_t_kernel,
        out_type=jax.ShapeDtypeStruct((nq,), jnp.float32),
        mesh=mesh,
        scratch_types=[
            pltpu.VMEM((_PADK,), jnp.float32),
            pltpu.VMEM((_NBUCKET,), jnp.int32),
            pltpu.VMEM((per_worker,), jnp.float32),
            pltpu.VMEM((per_worker,), jnp.float32),
            pltpu.SemaphoreType.DMA,
            pltpu.SemaphoreType.DMA,
        ],
        compiler_params=cp,
    )
    return run(sigma, sigmas).reshape(sigma.shape)
